# Initial kernel scaffold; baseline (speedup 1.0000x reference)
import jax
import jax.numpy as jnp
from jax.experimental import pallas as pl


def kernel(x, edge_index, params):
    raise NotImplementedError("write your pallas kernel here")


# speedup vs baseline: 9.8530x; 9.8530x over previous
import functools
import math

import jax
import jax.numpy as jnp
from jax import lax
from jax.experimental import pallas as pl
from jax.experimental.pallas import tpu as pltpu
from jax.experimental.pallas import tpu_sc as plsc

N = 10000
E = 320000
HID = 128
NC = 2
NS = 16
NW = NC * NS
CW = 128
CHUNKS = 79
EPW = CHUNKS * CW
EP = NW * EPW
NPAD = 10112
TRASH = 10000
ZROWS = NPAD // NS
RB = 1000
GB = N // RB
NP2 = CHUNKS * CW
BIGF = float(1 << 20)

_mesh = functools.partial(
    plsc.VectorSubcoreMesh, core_axis_name="c", subcore_axis_name="s",
    num_cores=NC)
_SC_PARAMS = pltpu.CompilerParams(needs_layout_passes=False)


def _sc_edge_valid_body(alive_hbm, src_hbm, dst_hbm, out_hbm,
                        alive_v, src_v, dst_v):
    c = lax.axis_index("c")
    s = lax.axis_index("s")
    w = c * NS + s
    pltpu.sync_copy(alive_hbm, alive_v)
    pltpu.sync_copy(src_hbm.at[w], src_v)
    pltpu.sync_copy(dst_hbm.at[w], dst_v)

    def chunk(j, carry):
        for t in range(CW // 16):
            sl = pl.ds(t * 16, 16)
            si = src_v[j, sl]
            di = dst_v[j, sl]
            a_s = plsc.load_gather(alive_v, [si])
            a_d = plsc.load_gather(alive_v, [di])
            ok = (a_s & a_d) > 0
            dst_v[j, sl] = jnp.where(ok, di, TRASH)
        return carry

    lax.fori_loop(0, CHUNKS, chunk, 0)
    pltpu.sync_copy(dst_v, out_hbm.at[w])


def _sc_edge_valid(alive_i32, src3, dst3):
    k = pl.kernel(
        _sc_edge_valid_body,
        out_type=jax.ShapeDtypeStruct((NW, CHUNKS, CW), jnp.int32),
        mesh=_mesh(), compiler_params=_SC_PARAMS,
        scratch_types=[
            pltpu.VMEM((NPAD,), jnp.int32),
            pltpu.VMEM((CHUNKS, CW), jnp.int32),
            pltpu.VMEM((CHUNKS, CW), jnp.int32),
        ],
    )
    return k(alive_i32, src3, dst3)


def _sc_degree_body(dst_hbm, ones_hbm, zeros_hbm, out_hbm,
                    idx_v, ones_v, acc):
    c = lax.axis_index("c")
    s = lax.axis_index("s")
    w = c * NS + s
    pltpu.sync_copy(dst_hbm.at[w], idx_v)
    pltpu.sync_copy(ones_hbm, ones_v)
    rows = pl.ds(s * ZROWS, ZROWS)
    pltpu.sync_copy(zeros_hbm, acc.at[rows])
    plsc.subcore_barrier()

    def chunk(j, carry):
        pltpu.sync_copy(ones_v, acc.at[idx_v.at[j]], add=True)
        return carry

    lax.fori_loop(0, CHUNKS, chunk, 0)
    plsc.subcore_barrier()
    pltpu.sync_copy(acc.at[rows], out_hbm.at[c, rows])


def _sc_degree(dst3, ones16, zeros16):
    k = pl.kernel(
        _sc_degree_body,
        out_type=jax.ShapeDtypeStruct((NC, NPAD, 16), jnp.float32),
        mesh=_mesh(),
        scratch_types=[
            pltpu.VMEM((CHUNKS, CW), jnp.int32),
            pltpu.VMEM((CW, 16), jnp.float32),
            pltpu.VMEM_SHARED((NPAD, 16), jnp.float32),
        ],
    )
    return k(dst3, ones16, zeros16)


def _sc_agg_body(hs_hbm, src_hbm, dst_hbm, zeros_hbm, out_hbm,
                 src_v, dst_v, rows_v, acc, sem):
    c = lax.axis_index("c")
    s = lax.axis_index("s")
    w = c * NS + s
    pltpu.sync_copy(src_hbm.at[w], src_v)
    pltpu.sync_copy(dst_hbm.at[w], dst_v)
    rows = pl.ds(s * ZROWS, ZROWS)
    pltpu.sync_copy(zeros_hbm, acc.at[rows])
    plsc.subcore_barrier()

    def chunk(j, carry):
        pltpu.async_copy(hs_hbm.at[src_v.at[j]], rows_v, sem).wait()
        pltpu.sync_copy(rows_v, acc.at[dst_v.at[j]], add=True)
        return carry

    lax.fori_loop(0, CHUNKS, chunk, 0)
    plsc.subcore_barrier()
    pltpu.sync_copy(acc.at[rows], out_hbm.at[c, rows])


def _sc_agg(hs, src3, dst3, zerosD):
    d = hs.shape[1]
    k = pl.kernel(
        _sc_agg_body,
        out_type=jax.ShapeDtypeStruct((NC, NPAD, d), jnp.float32),
        mesh=_mesh(),
        scratch_types=[
            pltpu.VMEM((CHUNKS, CW), jnp.int32),
            pltpu.VMEM((CHUNKS, CW), jnp.int32),
            pltpu.VMEM((CW, d), jnp.float32),
            pltpu.VMEM_SHARED((NPAD, d), jnp.float32),
            pltpu.SemaphoreType.DMA,
        ],
    )
    return k(hs, src3, dst3, zerosD)


def _mm_body(x_ref, w_ref, dis_ref, o_ref):
    h = lax.dot_general(x_ref[...], w_ref[...], (((1,), (0,)), ((), ())),
                        preferred_element_type=jnp.float32)
    o_ref[...] = h * dis_ref[...]


def _mm2_body(x_ref, x2_ref, w_ref, dis_ref, o_ref):
    h = lax.dot_general(x_ref[...] + x2_ref[...], w_ref[...],
                        (((1,), (0,)), ((), ())),
                        preferred_element_type=jnp.float32)
    o_ref[...] = h * dis_ref[...]


def _mm_scale(x, w, dis, x2=None, interp=False):
    d = w.shape[1]
    xspec = pl.BlockSpec((RB, HID), lambda i: (i, 0))
    specs = [xspec] + ([xspec] if x2 is not None else []) + [
        pl.BlockSpec((HID, d), lambda i: (0, 0)),
        pl.BlockSpec((RB, 1), lambda i: (i, 0)),
    ]
    body = _mm2_body if x2 is not None else _mm_body
    args = (x, x2, w, dis) if x2 is not None else (x, w, dis)
    return pl.pallas_call(
        body, grid=(GB,), in_specs=specs,
        out_specs=pl.BlockSpec((RB, d), lambda i: (i, 0)),
        out_shape=jax.ShapeDtypeStruct((N, d), jnp.float32),
        interpret=interp)(*args)


def _post_body(relu, p0_ref, p1_ref, hs_ref, dis_ref, b_ref, al_ref, o_ref):
    o = (p0_ref[...] + p1_ref[...] + hs_ref[...]) * dis_ref[...] + b_ref[...]
    if relu:
        o = jnp.maximum(o, 0.0)
    o_ref[...] = o * al_ref[...]


def _post(p0, p1, hs, dis, b, alivef, relu, interp=False):
    d = hs.shape[1]
    return pl.pallas_call(
        functools.partial(_post_body, relu),
        grid=(GB,),
        in_specs=[
            pl.BlockSpec((RB, d), lambda i: (i, 0)),
            pl.BlockSpec((RB, d), lambda i: (i, 0)),
            pl.BlockSpec((RB, d), lambda i: (i, 0)),
            pl.BlockSpec((RB, 1), lambda i: (i, 0)),
            pl.BlockSpec((1, d), lambda i: (0, 0)),
            pl.BlockSpec((RB, 1), lambda i: (i, 0)),
        ],
        out_specs=pl.BlockSpec((RB, d), lambda i: (i, 0)),
        out_shape=jax.ShapeDtypeStruct((N, d), jnp.float32),
        interpret=interp)(p0, p1, hs, dis, b, alivef)


def _dis_body(c0_ref, c1_ref, o_ref):
    cnt = c0_ref[...][:, 0:1] + c1_ref[...][:, 0:1]
    o_ref[...] = lax.rsqrt(1.0 + cnt)


def _dis_from_parts(c0, c1, interp=False):
    return pl.pallas_call(
        _dis_body, grid=(GB,),
        in_specs=[pl.BlockSpec((RB, 16), lambda i: (i, 0)),
                  pl.BlockSpec((RB, 16), lambda i: (i, 0))],
        out_specs=pl.BlockSpec((RB, 1), lambda i: (i, 0)),
        out_shape=jax.ShapeDtypeStruct((N, 1), jnp.float32),
        interpret=interp)(c0, c1)


def _score_body(x_ref, p_ref, al_ref, s_ref, sm_ref):
    p = p_ref[...]
    t = lax.dot_general(x_ref[...], p, (((1,), (0,)), ((), ())),
                        preferred_element_type=jnp.float32)
    nrm = jnp.sqrt(jnp.sum(p * p))
    s = jnp.tanh(t / nrm)
    s_ref[...] = s
    sm_ref[...] = jnp.where(al_ref[...] > 0, s, -2.0)


def _score(x, p, alivef, interp=False):
    return pl.pallas_call(
        _score_body, grid=(GB,),
        in_specs=[pl.BlockSpec((RB, HID), lambda i: (i, 0)),
                  pl.BlockSpec((HID, 1), lambda i: (0, 0)),
                  pl.BlockSpec((RB, 1), lambda i: (i, 0))],
        out_specs=[pl.BlockSpec((RB, 1), lambda i: (i, 0)),
                   pl.BlockSpec((RB, 1), lambda i: (i, 0))],
        out_shape=[jax.ShapeDtypeStruct((N, 1), jnp.float32),
                   jax.ShapeDtypeStruct((N, 1), jnp.float32)],
        interpret=interp)(x, p.reshape(HID, 1), alivef)


def _rank_body(sc_ref, rc_ref, sr_ref, rr_ref, o_ref):
    si = sc_ref[...]
    ri = rc_ref[...]

    def step(j, acc):
        sj = sr_ref[pl.ds(j, 1), :]
        rj = rr_ref[pl.ds(j, 1), :]
        gt = sj > si
        eq = (sj == si) & (rj < ri)
        return acc + jnp.where(gt | eq, 1.0, 0.0)

    acc = lax.fori_loop(0, CHUNKS, step,
                        jnp.zeros((CW, CW), jnp.float32))
    o_ref[...] = jnp.sum(acc, axis=1, keepdims=True)


def _rank(s_col, rk_col, s_row, rk_row, interp=False):
    return pl.pallas_call(
        _rank_body, grid=(CHUNKS,),
        in_specs=[pl.BlockSpec((CW, 1), lambda i: (i, 0)),
                  pl.BlockSpec((CW, 1), lambda i: (i, 0)),
                  pl.BlockSpec((CHUNKS, CW), lambda i: (0, 0)),
                  pl.BlockSpec((CHUNKS, CW), lambda i: (0, 0))],
        out_specs=pl.BlockSpec((CW, 1), lambda i: (i, 0)),
        out_shape=jax.ShapeDtypeStruct((NP2, 1), jnp.float32),
        interpret=interp)(s_col, rk_col, s_row, rk_row)


def _pool_body(kf, x_ref, s_ref, r_ref, al_ref, xo_ref, ko_ref, ro_ref):
    keep = (al_ref[...] > 0) & (r_ref[...] < kf)
    keepf = jnp.where(keep, 1.0, 0.0)
    xo_ref[...] = x_ref[...] * (s_ref[...] * keepf)
    ko_ref[...] = keepf
    ro_ref[...] = jnp.where(keep, r_ref[...], BIGF)


def _pool_update(x, s, rank, alivef, kf, interp=False):
    return pl.pallas_call(
        functools.partial(_pool_body, kf), grid=(GB,),
        in_specs=[pl.BlockSpec((RB, HID), lambda i: (i, 0)),
                  pl.BlockSpec((RB, 1), lambda i: (i, 0)),
                  pl.BlockSpec((RB, 1), lambda i: (i, 0)),
                  pl.BlockSpec((RB, 1), lambda i: (i, 0))],
        out_specs=[pl.BlockSpec((RB, HID), lambda i: (i, 0)),
                   pl.BlockSpec((RB, 1), lambda i: (i, 0)),
                   pl.BlockSpec((RB, 1), lambda i: (i, 0))],
        out_shape=[jax.ShapeDtypeStruct((N, HID), jnp.float32),
                   jax.ShapeDtypeStruct((N, 1), jnp.float32),
                   jax.ShapeDtypeStruct((N, 1), jnp.float32)],
        interpret=interp)(x, s, rank, alivef)


def _head_body(x_ref, w1_ref, b1_ref, w2_ref, b2_ref, wft_ref, bf_ref, o_ref):
    h = jnp.tanh(x_ref[...][:, 0:2])
    h1 = jnp.tanh(h[:, 0:1] * w1_ref[0:1, :] + h[:, 1:2] * w1_ref[1:2, :]
                  + b1_ref[...])
    acc = jnp.zeros((RB, 32), jnp.float32) + b2_ref[...]
    for t in range(16):
        acc = acc + h1[:, t:t + 1] * w2_ref[t:t + 1, :]
    h2 = jnp.tanh(acc)
    f0 = jnp.sum(h2 * wft_ref[0:1, :], axis=1, keepdims=True)
    f1 = jnp.sum(h2 * wft_ref[1:2, :], axis=1, keepdims=True)
    o_ref[...] = jnp.concatenate([f0, f1], axis=1) + bf_ref[...]


def _head(xf, w1, b1, w2, b2, wf, bf, interp=False):
    return pl.pallas_call(
        _head_body, grid=(GB,),
        in_specs=[pl.BlockSpec((RB, HID), lambda i: (i, 0)),
                  pl.BlockSpec((2, 16), lambda i: (0, 0)),
                  pl.BlockSpec((1, 16), lambda i: (0, 0)),
                  pl.BlockSpec((16, 32), lambda i: (0, 0)),
                  pl.BlockSpec((1, 32), lambda i: (0, 0)),
                  pl.BlockSpec((2, 32), lambda i: (0, 0)),
                  pl.BlockSpec((1, 2), lambda i: (0, 0))],
        out_specs=pl.BlockSpec((RB, 2), lambda i: (i, 0)),
        out_shape=jax.ShapeDtypeStruct((N, 2), jnp.float32),
        interpret=interp)(xf, w1, b1.reshape(1, 16), w2, b2.reshape(1, 32),
                          wf.T, bf.reshape(1, 2))


def _qr_body(a_ref, q_ref):
    a = a_ref[...]
    a1 = a[:, 0:1]
    a2 = a[:, 1:2]
    n1 = jnp.sum(a1 * a1)
    d12 = jnp.sum(a1 * a2)
    a11 = a_ref[0, 0]
    a21 = a_ref[1, 0]
    a12 = a_ref[0, 1]
    a22 = a_ref[1, 1]
    beta1 = jnp.where(a11 >= 0, -1.0, 1.0) * jnp.sqrt(n1)
    q1 = a1 / beta1
    r12 = d12 / beta1
    u = a2 - r12 * q1
    w1 = a22 - a21 * (d12 - beta1 * a12) / (beta1 * (beta1 - a11))
    beta2 = jnp.where(w1 >= 0, -1.0, 1.0) * jnp.sqrt(jnp.sum(u * u))
    q_ref[...] = jnp.concatenate([q1, u / beta2], axis=1)


def _qr(hf, interp=False):
    return pl.pallas_call(
        _qr_body,
        in_specs=[pl.BlockSpec((N, 2), lambda: (0, 0))],
        out_specs=pl.BlockSpec((N, 2), lambda: (0, 0)),
        out_shape=jax.ShapeDtypeStruct((N, 2), jnp.float32),
        interpret=interp)(hf)


def _gcn(x, w, b, dis, src3, dstm3, alivef, relu, zerosD, x2=None,
         interp=False):
    hs = _mm_scale(x, w, dis, x2=x2, interp=interp)
    parts = _sc_agg(hs, src3, dstm3, zerosD)
    d = w.shape[1]
    return _post(parts[0, :N], parts[1, :N], hs, dis,
                 b.reshape(1, d), alivef, relu, interp=interp)


def _level_setup(alivef, src3, dst3, ones16, zeros16, interp=False):
    alive_i32 = jnp.pad(alivef[:, 0].astype(jnp.int32), (0, NPAD - N))
    dstm3 = _sc_edge_valid(alive_i32, src3, dst3)
    cnt = _sc_degree(dstm3, ones16, zeros16)
    dis = _dis_from_parts(cnt[0, :N], cnt[1, :N], interp=interp)
    return dstm3, dis


def _pool(x, p, alivef, rord, kf, interp=False):
    s, sm = _score(x, p, alivef, interp=interp)
    s_col = jnp.pad(sm, ((0, NP2 - N), (0, 0)), constant_values=-2.0)
    r_col = jnp.pad(rord, ((0, NP2 - N), (0, 0)), constant_values=2.0 * BIGF)
    rank = _rank(s_col, r_col, s_col.reshape(CHUNKS, CW),
                 r_col.reshape(CHUNKS, CW), interp=interp)
    xn, keepf, rordn = _pool_update(x, s, rank[:N], alivef, kf,
                                    interp=interp)
    return xn, keepf, rordn


def _run(x, edge_index, params, interp=False):
    src = edge_index[0].astype(jnp.int32)
    dst = edge_index[1].astype(jnp.int32)
    src3 = jnp.pad(src, (0, EP - E)).reshape(NW, CHUNKS, CW)
    dst3 = jnp.pad(dst, (0, EP - E),
                   constant_values=TRASH).reshape(NW, CHUNKS, CW)
    ones16 = jnp.ones((CW, 16), jnp.float32)
    zeros16 = jnp.zeros((ZROWS, 16), jnp.float32)
    zeros128 = jnp.zeros((ZROWS, HID), jnp.float32)
    onesN = jnp.ones((N, 1), jnp.float32)
    rord0 = jnp.arange(N, dtype=jnp.float32).reshape(N, 1)

    kf = [float(math.ceil(0.95 * N))]
    kf.append(float(math.ceil(0.85 * kf[0])))
    kf.append(float(math.ceil(0.85 * kf[1])))

    dW, db = params['down_W'], params['down_b']
    uW, ub = params['up_W'], params['up_b']

    cnt0 = _sc_degree(dst3, ones16, zeros16)
    dis0 = _dis_from_parts(cnt0[0, :N], cnt0[1, :N], interp=interp)
    x0 = _gcn(x, dW[0], db[0], dis0, src3, dst3, onesN, True, zeros128,
              interp=interp)

    xs, diss, dsts, alives = [x0], [dis0], [dst3], [onesN]
    xc, alivef, rord = x0, onesN, rord0
    for lvl in range(1, 4):
        xc, alivef, rord = _pool(xc, params['pool_p'][lvl - 1], alivef, rord,
                                 kf[lvl - 1], interp=interp)
        dstm3, dis = _level_setup(alivef, src3, dst3, ones16, zeros16,
                                  interp=interp)
        xc = _gcn(xc, dW[lvl], db[lvl], dis, src3, dstm3, alivef, True,
                  zeros128, interp=interp)
        if lvl < 3:
            xs.append(xc)
            diss.append(dis)
            dsts.append(dstm3)
            alives.append(alivef)

    for i in range(3):
        j = 2 - i
        if j > 0:
            xc = _gcn(xc, uW[i], ub[i], diss[j], src3, dsts[j], alives[j],
                      True, zeros128, x2=xs[j], interp=interp)
        else:
            w2p = jnp.pad(uW[i], ((0, 0), (0, HID - 2)))
            b2p = jnp.pad(ub[i], (0, HID - 2))
            xc = _gcn(xc, w2p, b2p, diss[0], src3, dsts[0], alives[0],
                      False, zeros128, x2=xs[0], interp=interp)

    hf = _head(xc, params['W1'], params['b1'], params['W2'], params['b2'],
               params['Wf'], params['bf'], interp=interp)
    return _qr(hf, interp=interp)


def kernel(x, edge_index, params):
    return _run(x, edge_index, params)
